# Initial kernel scaffold; baseline (speedup 1.0000x reference)
import jax
import jax.numpy as jnp
from jax.experimental import pallas as pl


def kernel(x1, edge_index, edge_weight, W1_rel, b1_rel, W1_root, gamma1, beta1, W2_rel, b2_rel, W2_root):
    raise NotImplementedError("write your pallas kernel here")


# speedup vs baseline: 4.0360x; 4.0360x over previous
import functools

import jax
import jax.numpy as jnp
from jax import lax
from jax.experimental import pallas as pl
from jax.experimental.pallas import tpu as pltpu
from jax.experimental.pallas import tpu_sc as plsc

N = 10000
E = 320000
D_IN = 128
D_HID = 256
D_OUT = 128
EPS = 1e-5

NC = 2
NS = 16
NW = NC * NS
B = 128
NB = -(-E // (NW * B))
PW = NB * B
EP = NW * PW
N_ACC = 10112
ROWS_PT = N_ACC // NS


def _segment_sum_sc(table, src_p, dst_p, w_p):
    mesh = plsc.VectorSubcoreMesh(core_axis_name="c", subcore_axis_name="s")

    @functools.partial(
        pl.kernel,
        out_type=jax.ShapeDtypeStruct((NC * N_ACC, D_IN), jnp.float32),
        mesh=mesh,
        scratch_types=[
            pltpu.VMEM_SHARED((N_ACC, D_IN), jnp.float32),
            pltpu.VMEM((B, D_IN), jnp.float32),
            pltpu.VMEM((B,), jnp.int32),
            pltpu.VMEM((B,), jnp.int32),
            pltpu.VMEM((B,), jnp.float32),
            pltpu.SemaphoreType.DMA,
        ],
    )
    def seg_kernel(table_h, src_h, dst_h, w_h, out_h, accum, rows, sidx, didx, wv, sem):
        c = lax.axis_index("c")
        s = lax.axis_index("s")
        wid = s * NC + c

        zero16 = jnp.zeros((16,), jnp.float32)

        def zero_body(b, carry):
            for k in range(D_IN // 16):
                rows[b, pl.ds(k * 16, 16)] = zero16
            return carry

        lax.fori_loop(0, B, zero_body, 0)

        full, rem = divmod(ROWS_PT, B)
        for j in range(full):
            pltpu.sync_copy(rows, accum.at[pl.ds(s * ROWS_PT + j * B, B)])
        if rem:
            pltpu.sync_copy(rows.at[pl.ds(0, rem)],
                            accum.at[pl.ds(s * ROWS_PT + full * B, rem)])
        plsc.subcore_barrier()

        def batch_body(g, carry):
            base = wid * PW + g * B
            pltpu.sync_copy(src_h.at[pl.ds(base, B)], sidx)
            pltpu.sync_copy(dst_h.at[pl.ds(base, B)], didx)
            pltpu.sync_copy(w_h.at[pl.ds(base, B)], wv)
            pltpu.async_copy(table_h.at[sidx], rows, sem).wait()

            def scale_body(g16, carry2):
                wvec = wv[pl.ds(g16 * 16, 16)]
                for l in range(16):
                    wt = wvec[l]
                    b = g16 * 16 + l
                    for k in range(D_IN // 16):
                        rows[b, pl.ds(k * 16, 16)] = rows[b, pl.ds(k * 16, 16)] * wt
                return carry2

            lax.fori_loop(0, B // 16, scale_body, 0)
            pltpu.sync_copy(rows, accum.at[didx], add=True)
            return carry

        lax.fori_loop(0, PW // B, batch_body, 0)
        plsc.subcore_barrier()

        pltpu.sync_copy(accum.at[pl.ds(s * ROWS_PT, ROWS_PT)],
                        out_h.at[pl.ds(c * N_ACC + s * ROWS_PT, ROWS_PT)])

    return seg_kernel(table, src_p, dst_p, w_p)


def _dense_stage(partials, x, W1_rel, b1, W1_root, gamma1, beta1, W2_rel, b2, W2_root):
    def body(p_ref, x_ref, w1r_ref, b1_ref, w1o_ref, g1_ref, be1_ref,
             w2r_ref, b2_ref, w2o_ref, hp_ref, root2_ref):
        agg = p_ref[0] + p_ref[1]
        h = lax.dot_general(agg, w1r_ref[...], (((1,), (1,)), ((), ())),
                            preferred_element_type=jnp.float32)
        h = h + lax.dot_general(x_ref[...], w1o_ref[...], (((1,), (1,)), ((), ())),
                                preferred_element_type=jnp.float32)
        h = h + b1_ref[...]
        mean = jnp.mean(h, axis=0, keepdims=True)
        var = jnp.mean((h - mean) ** 2, axis=0, keepdims=True)
        hn = (h - mean) * lax.rsqrt(var + EPS) * g1_ref[...] + be1_ref[...]
        hn = jnp.maximum(hn, 0.0)
        hp_ref[...] = lax.dot_general(hn, w2r_ref[...], (((1,), (1,)), ((), ())),
                                      preferred_element_type=jnp.float32)
        root2_ref[...] = lax.dot_general(hn, w2o_ref[...], (((1,), (1,)), ((), ())),
                                         preferred_element_type=jnp.float32) + b2_ref[...]

    return pl.pallas_call(
        body,
        out_shape=[
            jax.ShapeDtypeStruct((N, D_OUT), jnp.float32),
            jax.ShapeDtypeStruct((N, D_OUT), jnp.float32),
        ],
    )(partials, x, W1_rel, b1.reshape(1, D_HID), W1_root,
      gamma1.reshape(1, D_HID), beta1.reshape(1, D_HID), W2_rel,
      b2.reshape(1, D_OUT), W2_root)


def _final_add(partials, root2):
    def body(p_ref, r_ref, o_ref):
        o_ref[...] = p_ref[0] + p_ref[1] + r_ref[...]

    return pl.pallas_call(
        body,
        out_shape=jax.ShapeDtypeStruct((N, D_OUT), jnp.float32),
    )(partials, root2)


def kernel(x1, edge_index, edge_weight, W1_rel, b1_rel, W1_root, gamma1, beta1,
           W2_rel, b2_rel, W2_root):
    src = edge_index[0]
    dst = edge_index[1]
    pad = EP - E
    src_p = jnp.concatenate([src, jnp.zeros((pad,), jnp.int32)])
    dst_p = jnp.concatenate([dst, jnp.zeros((pad,), jnp.int32)])
    w_p = jnp.concatenate([edge_weight, jnp.zeros((pad,), jnp.float32)])

    p1 = _segment_sum_sc(x1, src_p, dst_p, w_p).reshape(NC, N_ACC, D_IN)[:, :N]
    hp, root2 = _dense_stage(p1, x1, W1_rel, b1_rel, W1_root, gamma1, beta1,
                             W2_rel, b2_rel, W2_root)
    p2 = _segment_sum_sc(hp, src_p, dst_p, w_p).reshape(NC, N_ACC, D_IN)[:, :N]
    return _final_add(p2, root2)
